# Initial kernel scaffold; baseline (speedup 1.0000x reference)
import jax
import jax.numpy as jnp
from jax.experimental import pallas as pl


def kernel(x, node_adj, gcn_w, e_w1, e_b1, e_w2, e_b2, n_w, n_b, w_ih, w_hh, b_ih, b_hh, fo_w, fo_b):
    raise NotImplementedError("write your pallas kernel here")


# speedup vs baseline: 198.2613x; 198.2613x over previous
import functools

import jax
import jax.numpy as jnp
from jax.experimental import pallas as pl


def _prep_body(x_ref, adj_ref, gcn_w_ref, w1s_ref, w1t_ref,
               h_ref, a_ref, c_ref):
    support = jnp.dot(x_ref[0], gcn_w_ref[...],
                      preferred_element_type=jnp.float32)
    h = jax.nn.relu(jnp.dot(adj_ref[...], support,
                            preferred_element_type=jnp.float32))
    h_ref[0] = h
    a_ref[0] = jnp.dot(h, w1s_ref[...], preferred_element_type=jnp.float32)
    c_ref[0] = jnp.dot(h, w1t_ref[...], preferred_element_type=jnp.float32)


def _edge_body(a_ref, c_ref, adj_ref, v_ref, b1_ref, w2t_ref, b2_ref,
               sub_ref, add_ref, *, ti, n):
    it = pl.program_id(1)
    a = a_ref[0]
    c = c_ref[0]
    adjb = adj_ref[...]
    x1 = (a[:, None, :] + c[None, :, :]
          + adjb[:, :, None] * v_ref[0][None, None, :]
          + b1_ref[0][None, None, :])
    s1 = jax.nn.sigmoid(x1).reshape(ti * n, 32)
    o2 = jnp.dot(s1, w2t_ref[...], preferred_element_type=jnp.float32)
    s2 = jax.nn.sigmoid(o2 + b2_ref[0])
    maskf = jnp.where(adjb != 0.0, 1.0, 0.0)
    m2 = s2.reshape(ti, n, 30) * maskf[:, :, None]
    sub_ref[0] = jnp.sum(m2, axis=1)
    colsum = jnp.sum(m2, axis=0)

    @pl.when(it == 0)
    def _():
        add_ref[0] = colsum

    @pl.when(it != 0)
    def _():
        add_ref[0] = add_ref[0] + colsum


def _head_body(add_ref, sub_ref, h_ref, nwt_ref, nb_ref,
               wg_ref, wx_ref, bih_ref, bhh_ref, fot_ref, fob_ref,
               out_ref, *, bn):
    agg = (add_ref[...] - sub_ref[...]).reshape(bn, 30)
    xg = jax.nn.sigmoid(jnp.dot(agg, nwt_ref[...],
                                preferred_element_type=jnp.float32)
                        + nb_ref[0])
    hflat = h_ref[...].reshape(bn, 128)
    gi = (jnp.dot(xg, wg_ref[...], preferred_element_type=jnp.float32)
          + jnp.dot(hflat, wx_ref[...], preferred_element_type=jnp.float32)
          + bih_ref[0])
    bhh = bhh_ref[0]
    i_r = gi[:, 0:64]
    i_z = gi[:, 64:128]
    i_n = gi[:, 128:192]
    r = jax.nn.sigmoid(i_r + bhh[0:64])
    z = jax.nn.sigmoid(i_z + bhh[64:128])
    nng = jnp.tanh(i_n + r * bhh[128:192])
    hn = (1.0 - z) * nng
    out_ref[...] = (jnp.dot(hn, fot_ref[...],
                            preferred_element_type=jnp.float32)
                    + fob_ref[0])


def kernel(x, node_adj, gcn_w, e_w1, e_b1, e_w2, e_b2, n_w, n_b,
           w_ih, w_hh, b_ih, b_hh, fo_w, fo_b):
    B, N, NF = x.shape
    EH = e_w1.shape[0]
    EO = e_w2.shape[0]
    GO = n_w.shape[0]
    HID = w_hh.shape[1]
    TI = 64
    NT = N // TI

    w1s = e_w1[:, :NF].T
    w1t = e_w1[:, NF:2 * NF].T
    v = e_w1[:, 2 * NF].reshape(1, EH)
    b1 = e_b1.reshape(1, EH)
    w2t = e_w2.T
    b2 = e_b2.reshape(1, EO)
    nwt = n_w.T
    nb = n_b.reshape(1, GO)
    wg = w_ih[:, :GO].T
    wx = w_ih[:, GO:].T
    bih = b_ih.reshape(1, 3 * HID)
    bhh = b_hh.reshape(1, 3 * HID)
    fot = fo_w.T
    fob = fo_b.reshape(1, 1)

    h, a, c = pl.pallas_call(
        _prep_body,
        grid=(B,),
        in_specs=[
            pl.BlockSpec((1, N, NF), lambda b: (b, 0, 0)),
            pl.BlockSpec((N, N), lambda b: (0, 0)),
            pl.BlockSpec((NF, NF), lambda b: (0, 0)),
            pl.BlockSpec((NF, EH), lambda b: (0, 0)),
            pl.BlockSpec((NF, EH), lambda b: (0, 0)),
        ],
        out_specs=[
            pl.BlockSpec((1, N, NF), lambda b: (b, 0, 0)),
            pl.BlockSpec((1, N, EH), lambda b: (b, 0, 0)),
            pl.BlockSpec((1, N, EH), lambda b: (b, 0, 0)),
        ],
        out_shape=[
            jax.ShapeDtypeStruct((B, N, NF), jnp.float32),
            jax.ShapeDtypeStruct((B, N, EH), jnp.float32),
            jax.ShapeDtypeStruct((B, N, EH), jnp.float32),
        ],
    )(x, node_adj, gcn_w, w1s, w1t)

    sub, add = pl.pallas_call(
        functools.partial(_edge_body, ti=TI, n=N),
        grid=(B, NT),
        in_specs=[
            pl.BlockSpec((1, TI, EH), lambda b, it: (b, it, 0)),
            pl.BlockSpec((1, N, EH), lambda b, it: (b, 0, 0)),
            pl.BlockSpec((TI, N), lambda b, it: (it, 0)),
            pl.BlockSpec((1, EH), lambda b, it: (0, 0)),
            pl.BlockSpec((1, EH), lambda b, it: (0, 0)),
            pl.BlockSpec((EH, EO), lambda b, it: (0, 0)),
            pl.BlockSpec((1, EO), lambda b, it: (0, 0)),
        ],
        out_specs=[
            pl.BlockSpec((1, TI, EO), lambda b, it: (b, it, 0)),
            pl.BlockSpec((1, N, EO), lambda b, it: (b, 0, 0)),
        ],
        out_shape=[
            jax.ShapeDtypeStruct((B, N, EO), jnp.float32),
            jax.ShapeDtypeStruct((B, N, EO), jnp.float32),
        ],
    )(a, c, node_adj, v, b1, w2t, b2)

    out = pl.pallas_call(
        functools.partial(_head_body, bn=B * N),
        in_specs=[pl.BlockSpec(arr.shape,
                               functools.partial(lambda nd: (0,) * nd,
                                                 arr.ndim))
                  for arr in (add, sub, h, nwt, nb, wg, wx, bih, bhh,
                              fot, fob)],
        out_specs=pl.BlockSpec((B * N, 1), lambda: (0, 0)),
        out_shape=jax.ShapeDtypeStruct((B * N, 1), jnp.float32),
    )(add, sub, h, nwt, nb, wg, wx, bih, bhh, fot, fob)

    return out.reshape(B, N, 1)[:, None, :, :]


# speedup vs baseline: 482.4702x; 2.4335x over previous
import functools

import jax
import jax.numpy as jnp
from jax.experimental import pallas as pl


def _prep_body(x_ref, adj_ref, gcn_w_ref, w1s_ref, w1t4_ref, b1t4_ref,
               h_ref, a_ref, c4b_ref):
    support = jnp.dot(x_ref[0], gcn_w_ref[...],
                      preferred_element_type=jnp.float32)
    h = jax.nn.relu(jnp.dot(adj_ref[...], support,
                            preferred_element_type=jnp.float32))
    h_ref[0] = h
    a_ref[0] = jnp.dot(h, w1s_ref[...], preferred_element_type=jnp.float32)
    c4b_ref[0] = (jnp.dot(h, w1t4_ref[...],
                          preferred_element_type=jnp.float32)
                  + b1t4_ref[...])


def _edge_body(a4_ref, c4b_ref, adjt_ref, rv_ref, w2b_ref, b2t_ref, rm_ref,
               sub_ref, acc_ref, *, gb, n):
    it = pl.program_id(1)
    flat4 = adjt_ref[...].reshape(gb * n, 4)
    term = jnp.dot(flat4, rv_ref[...], preferred_element_type=jnp.float32)
    x1 = (term.reshape(gb, n, 128)
          + a4_ref[0][:, None, :]
          + c4b_ref[0][None, :, :])
    s1 = jax.nn.sigmoid(x1).reshape(gb * n, 128)
    o2 = (jnp.dot(s1, w2b_ref[...], preferred_element_type=jnp.float32)
          + b2t_ref[...])
    s2 = jax.nn.sigmoid(o2)
    mf = jnp.where(flat4 != 0.0, 1.0, 0.0)
    mm = jnp.dot(mf, rm_ref[...], preferred_element_type=jnp.float32)
    m2 = (s2 * mm).reshape(gb, n, 120)
    sub_ref[0] = jnp.sum(m2, axis=1)
    colsum = jnp.sum(m2, axis=0)

    @pl.when(it == 0)
    def _():
        acc_ref[0] = colsum

    @pl.when(it != 0)
    def _():
        acc_ref[0] = acc_ref[0] + colsum


def _head_body(acc_ref, sub_ref, h_ref, fold_ref, nwt_ref, nb_ref,
               wg_ref, wx_ref, bih_ref, bhh_ref, fot_ref, fob_ref,
               out_ref, *, bn):
    addf = jnp.dot(acc_ref[...].reshape(bn, 120), fold_ref[...],
                   preferred_element_type=jnp.float32)
    agg = addf - sub_ref[...].reshape(bn, 30)
    xg = jax.nn.sigmoid(jnp.dot(agg, nwt_ref[...],
                                preferred_element_type=jnp.float32)
                        + nb_ref[...])
    hflat = h_ref[...].reshape(bn, 128)
    gi = (jnp.dot(xg, wg_ref[...], preferred_element_type=jnp.float32)
          + jnp.dot(hflat, wx_ref[...], preferred_element_type=jnp.float32)
          + bih_ref[...])
    bhh = bhh_ref[...]
    r = jax.nn.sigmoid(gi[:, 0:64] + bhh[:, 0:64])
    z = jax.nn.sigmoid(gi[:, 64:128] + bhh[:, 64:128])
    nng = jnp.tanh(gi[:, 128:192] + r * bhh[:, 128:192])
    hn = (1.0 - z) * nng
    out_ref[...] = (jnp.dot(hn, fot_ref[...],
                            preferred_element_type=jnp.float32)
                    + fob_ref[...])


def kernel(x, node_adj, gcn_w, e_w1, e_b1, e_w2, e_b2, n_w, n_b,
           w_ih, w_hh, b_ih, b_hh, fo_w, fo_b):
    B, N, NF = x.shape
    EH = e_w1.shape[0]
    EO = e_w2.shape[0]
    GO = n_w.shape[0]
    HID = w_hh.shape[1]
    G = N // 4
    GB = 32
    NT = G // GB

    w1s = e_w1[:, :NF].T
    w1t4 = jnp.tile(e_w1[:, NF:2 * NF].T, (1, 4))
    b1t4 = jnp.tile(e_b1.reshape(1, EH), (1, 4))
    v = e_w1[:, 2 * NF].reshape(1, EH)
    rv = jnp.kron(jnp.eye(4, dtype=x.dtype), v)
    rm = jnp.kron(jnp.eye(4, dtype=x.dtype),
                  jnp.ones((1, EO), x.dtype))
    w2b = jnp.kron(jnp.eye(4, dtype=x.dtype), e_w2.T)
    b2t = jnp.tile(e_b2.reshape(1, EO), (1, 4))
    fold = jnp.kron(jnp.ones((4, 1), x.dtype),
                    jnp.eye(EO, dtype=x.dtype))
    nwt = n_w.T
    nb = n_b.reshape(1, GO)
    wg = w_ih[:, :GO].T
    wx = w_ih[:, GO:].T
    bih = b_ih.reshape(1, 3 * HID)
    bhh = b_hh.reshape(1, 3 * HID)
    fot = fo_w.T
    fob = fo_b.reshape(1, 1)
    adjt = node_adj.reshape(G, 4, N).transpose(0, 2, 1)

    h, a, c4b = pl.pallas_call(
        _prep_body,
        grid=(B,),
        in_specs=[
            pl.BlockSpec((1, N, NF), lambda b: (b, 0, 0)),
            pl.BlockSpec((N, N), lambda b: (0, 0)),
            pl.BlockSpec((NF, NF), lambda b: (0, 0)),
            pl.BlockSpec((NF, EH), lambda b: (0, 0)),
            pl.BlockSpec((NF, 4 * EH), lambda b: (0, 0)),
            pl.BlockSpec((1, 4 * EH), lambda b: (0, 0)),
        ],
        out_specs=[
            pl.BlockSpec((1, N, NF), lambda b: (b, 0, 0)),
            pl.BlockSpec((1, N, EH), lambda b: (b, 0, 0)),
            pl.BlockSpec((1, N, 4 * EH), lambda b: (b, 0, 0)),
        ],
        out_shape=[
            jax.ShapeDtypeStruct((B, N, NF), jnp.float32),
            jax.ShapeDtypeStruct((B, N, EH), jnp.float32),
            jax.ShapeDtypeStruct((B, N, 4 * EH), jnp.float32),
        ],
    )(x, node_adj, gcn_w, w1s, w1t4, b1t4)

    a4 = a.reshape(B, G, 4 * EH)

    sub, acc = pl.pallas_call(
        functools.partial(_edge_body, gb=GB, n=N),
        grid=(B, NT),
        in_specs=[
            pl.BlockSpec((1, GB, 4 * EH), lambda b, it: (b, it, 0)),
            pl.BlockSpec((1, N, 4 * EH), lambda b, it: (b, 0, 0)),
            pl.BlockSpec((GB, N, 4), lambda b, it: (it, 0, 0)),
            pl.BlockSpec((4, 4 * EH), lambda b, it: (0, 0)),
            pl.BlockSpec((4 * EH, 4 * EO), lambda b, it: (0, 0)),
            pl.BlockSpec((1, 4 * EO), lambda b, it: (0, 0)),
            pl.BlockSpec((4, 4 * EO), lambda b, it: (0, 0)),
        ],
        out_specs=[
            pl.BlockSpec((1, GB, 4 * EO), lambda b, it: (b, it, 0)),
            pl.BlockSpec((1, N, 4 * EO), lambda b, it: (b, 0, 0)),
        ],
        out_shape=[
            jax.ShapeDtypeStruct((B, G, 4 * EO), jnp.float32),
            jax.ShapeDtypeStruct((B, N, 4 * EO), jnp.float32),
        ],
    )(a4, c4b, adjt, rv, w2b, b2t, rm)

    subn = sub.reshape(B, N, EO)

    out = pl.pallas_call(
        functools.partial(_head_body, bn=B * N),
        in_specs=[pl.BlockSpec(arr.shape,
                               functools.partial(lambda nd: (0,) * nd,
                                                 arr.ndim))
                  for arr in (acc, subn, h, fold, nwt, nb, wg, wx, bih,
                              bhh, fot, fob)],
        out_specs=pl.BlockSpec((B * N, 1), lambda: (0, 0)),
        out_shape=jax.ShapeDtypeStruct((B * N, 1), jnp.float32),
    )(acc, subn, h, fold, nwt, nb, wg, wx, bih, bhh, fot, fob)

    return out.reshape(B, N, 1)[:, None, :, :]
